# Initial kernel scaffold; baseline (speedup 1.0000x reference)
import jax
import jax.numpy as jnp
from jax.experimental import pallas as pl


def kernel(inputs, embedding):
    raise NotImplementedError("write your pallas kernel here")


# speedup vs baseline: 1.2718x; 1.2718x over previous
import jax
import jax.numpy as jnp
from jax.experimental import pallas as pl
from jax.experimental.pallas import tpu as pltpu

NUM_EMB = 8192
DIM = 32
ROWS = 16384
BLK = 128
GRID = ROWS // BLK
COMMIT = 0.25


def _vq_block(x_ref, emb_ref, xn_ref, en_ref,
              enc_ref, qst_ref, loss_ref, perp_ref, counts_ref):
    i = pl.program_id(0)
    x = x_ref[...]
    emb = emb_ref[...]
    mm = jax.lax.dot_general(x, emb, (((1,), (1,)), ((), ())),
                             preferred_element_type=jnp.float32)
    xn = xn_ref[...]
    en = en_ref[...]
    d = (xn + en) - 2.0 * mm
    HALF = NUM_EMB // 2
    d0 = d[:, :HALF]
    d1 = d[:, HALF:]
    m0 = jnp.min(d0, axis=1, keepdims=True)
    m1 = jnp.min(d1, axis=1, keepdims=True)
    colh = jax.lax.broadcasted_iota(jnp.int32, (BLK, HALF), 1)
    i0 = jnp.min(jnp.where(d0 == m0, colh, HALF), axis=1, keepdims=True)
    i1 = jnp.min(jnp.where(d1 == m1, colh, HALF), axis=1, keepdims=True) + HALF
    win1 = m1 < m0.astype(jnp.bfloat16).astype(jnp.float32)
    idx = jnp.where(win1, i1, i0)
    col = jax.lax.broadcasted_iota(jnp.int32, (BLK, NUM_EMB), 1)
    enc = jnp.where(col == idx, 1.0, 0.0).astype(jnp.float32)
    enc_ref[...] = enc
    ebr = emb.astype(jnp.bfloat16).astype(jnp.float32)
    q = jax.lax.dot_general(enc, ebr, (((1,), (0,)), ((), ())),
                            preferred_element_type=jnp.float32)
    qst_ref[...] = x + (q - x)
    lm = jnp.mean((q - x) ** 2, axis=1, keepdims=True)
    loss_ref[...] = lm + COMMIT * lm

    @pl.when(i == 0)
    def _init():
        counts_ref[...] = jnp.zeros_like(counts_ref)

    counts_ref[...] += jnp.sum(enc, axis=0, keepdims=True)

    @pl.when(i == GRID - 1)
    def _finish():
        avg = counts_ref[...] * (1.0 / ROWS)
        s = jnp.sum(avg * jnp.log(avg + 1e-10), axis=1, keepdims=True)
        perp_ref[...] = jnp.exp(-s)


def kernel(inputs, embedding):
    x = jnp.transpose(inputs, (0, 2, 3, 1))
    in_shape = x.shape
    flat = x.reshape(ROWS, DIM)
    xn = jnp.sum(flat ** 2, axis=1, keepdims=True)
    en = jnp.sum(embedding ** 2, axis=1).reshape(1, NUM_EMB)

    enc, qst, loss2d, perp = pl.pallas_call(
        _vq_block,
        grid=(GRID,),
        in_specs=[
            pl.BlockSpec((BLK, DIM), lambda i: (i, 0)),
            pl.BlockSpec((NUM_EMB, DIM), lambda i: (0, 0)),
            pl.BlockSpec((BLK, 1), lambda i: (i, 0)),
            pl.BlockSpec((1, NUM_EMB), lambda i: (0, 0)),
        ],
        out_specs=[
            pl.BlockSpec((BLK, NUM_EMB), lambda i: (i, 0)),
            pl.BlockSpec((BLK, DIM), lambda i: (i, 0)),
            pl.BlockSpec((BLK, 1), lambda i: (i, 0)),
            pl.BlockSpec((1, 1), lambda i: (0, 0)),
        ],
        out_shape=[
            jax.ShapeDtypeStruct((ROWS, NUM_EMB), jnp.float32),
            jax.ShapeDtypeStruct((ROWS, DIM), jnp.float32),
            jax.ShapeDtypeStruct((ROWS, 1), jnp.float32),
            jax.ShapeDtypeStruct((1, 1), jnp.float32),
        ],
        scratch_shapes=[pltpu.VMEM((1, NUM_EMB), jnp.float32)],
        compiler_params=pltpu.CompilerParams(
            dimension_semantics=("arbitrary",)),
    )(flat, embedding, xn, en)

    quantized_out = jnp.transpose(qst.reshape(in_shape), (0, 3, 1, 2))
    loss = loss2d.reshape(in_shape[0], in_shape[1], in_shape[2])
    return (loss, quantized_out, perp[0, 0], enc)
